# Initial kernel scaffold; baseline (speedup 1.0000x reference)
import jax
import jax.numpy as jnp
from jax.experimental import pallas as pl


def kernel(x_A, herb_batch_A, x_B, herb_batch_B, W_gnn, W_attn):
    raise NotImplementedError("write your pallas kernel here")


# speedup vs baseline: 15.4724x; 15.4724x over previous
import functools

import jax
import jax.numpy as jnp
from jax import lax
from jax.experimental import pallas as pl
from jax.experimental.pallas import tpu as pltpu

N = 100000
D = 128
S = 256
R = 2000
NB = N // R
NEG = -1e30

_INTERPRET = False


def _onehot(seg, dtype=jnp.float32):
    cols = lax.broadcasted_iota(jnp.int32, (R, S), 1)
    return jnp.where(seg[:, None] == cols, jnp.array(1.0, dtype), jnp.array(0.0, dtype))


def _pass1_body(xa_ref, sa_ref, xb_ref, sb_ref, sums_ref, cnts_ref):
    i = pl.program_id(0)

    @pl.when(i == 0)
    def _():
        sums_ref[...] = jnp.zeros_like(sums_ref)
        cnts_ref[...] = jnp.zeros_like(cnts_ref)

    for side, (x_ref, s_ref) in enumerate(((xa_ref, sa_ref), (xb_ref, sb_ref))):
        seg = s_ref[0, 0, :]
        oh = _onehot(seg)
        x = x_ref[...]
        sums_ref[side] += lax.dot_general(oh, x, (((0,), (0,)), ((), ())),
                                          preferred_element_type=jnp.float32)
        cnts_ref[side] += jnp.sum(oh, axis=0)


def _tiny_g_body(sums_ref, cnts_ref, wg_ref, wa_ref, g_ref):
    wg = wg_ref[...]
    wa = wa_ref[...]
    m1 = jnp.dot(wg, wa, preferred_element_type=jnp.float32)
    M = jnp.dot(m1, wg.T, preferred_element_type=jnp.float32)
    cnt = jnp.maximum(cnts_ref[...], 1.0)
    mean = sums_ref[...] / cnt[:, :, None]
    g_ref[0] = jnp.dot(mean[1], M, preferred_element_type=jnp.float32)
    g_ref[1] = jnp.dot(mean[0], M, preferred_element_type=jnp.float32)


def _pass2_body(xa_ref, sa_ref, xb_ref, sb_ref, g_ref, u_ref, m_ref, d_ref):
    i = pl.program_id(0)

    @pl.when(i == 0)
    def _():
        u_ref[...] = jnp.zeros_like(u_ref)
        m_ref[...] = jnp.full_like(m_ref, NEG)
        d_ref[...] = jnp.zeros_like(d_ref)

    for side, (x_ref, s_ref) in enumerate(((xa_ref, sa_ref), (xb_ref, sb_ref))):
        seg = s_ref[0, 0, :]
        oh = _onehot(seg)
        ohb = seg[:, None] == lax.broadcasted_iota(jnp.int32, (R, S), 1)
        x = x_ref[...]
        g = lax.dot_general(oh, g_ref[side], (((1,), (0,)), ((), ())),
                            preferred_element_type=jnp.float32)
        e = jnp.sum(x * g, axis=1)
        me = jnp.max(jnp.where(ohb, e[:, None], NEG), axis=0)
        m_old = m_ref[side]
        m_new = jnp.maximum(m_old, me)
        m_gath = jnp.sum(oh * m_new[None, :], axis=1)
        w = jnp.exp(e - m_gath)
        scale = jnp.exp(m_old - m_new)
        d_ref[side] = d_ref[side] * scale + jnp.sum(oh * w[:, None], axis=0)
        wx = x * w[:, None]
        u_ref[side] = (u_ref[side] * scale[:, None]
                       + lax.dot_general(oh, wx, (((0,), (0,)), ((), ())),
                                         preferred_element_type=jnp.float32))
        m_ref[side] = m_new


def _final_body(u_ref, d_ref, wg_ref, outa_ref, outb_ref):
    wg = wg_ref[...]
    den = d_ref[...] + 1e-16
    pooled = u_ref[...] / den[:, :, None]
    outa_ref[...] = jnp.dot(pooled[0], wg, preferred_element_type=jnp.float32)
    outb_ref[...] = jnp.dot(pooled[1], wg, preferred_element_type=jnp.float32)


def kernel(x_A, herb_batch_A, x_B, herb_batch_B, W_gnn, W_attn):
    segA = herb_batch_A.astype(jnp.int32).reshape(NB, 1, R)
    segB = herb_batch_B.astype(jnp.int32).reshape(NB, 1, R)

    xspec = pl.BlockSpec((R, D), lambda i: (i, 0))
    sspec = pl.BlockSpec((1, 1, R), lambda i: (i, 0, 0))
    full2 = pl.BlockSpec((2, S, D), lambda i: (0, 0, 0))
    full1 = pl.BlockSpec((2, S), lambda i: (0, 0))

    sums, cnts = pl.pallas_call(
        _pass1_body,
        grid=(NB,),
        in_specs=[xspec, sspec, xspec, sspec],
        out_specs=[full2, full1],
        out_shape=[jax.ShapeDtypeStruct((2, S, D), jnp.float32),
                   jax.ShapeDtypeStruct((2, S), jnp.float32)],
        compiler_params=pltpu.CompilerParams(
            dimension_semantics=("arbitrary",)),
        interpret=_INTERPRET,
    )(x_A, segA, x_B, segB)

    G = pl.pallas_call(
        _tiny_g_body,
        out_shape=jax.ShapeDtypeStruct((2, S, D), jnp.float32),
        interpret=_INTERPRET,
    )(sums, cnts, W_gnn, W_attn)

    U, m, d = pl.pallas_call(
        _pass2_body,
        grid=(NB,),
        in_specs=[xspec, sspec, xspec, sspec, full2],
        out_specs=[full2, full1, full1],
        out_shape=[jax.ShapeDtypeStruct((2, S, D), jnp.float32),
                   jax.ShapeDtypeStruct((2, S), jnp.float32),
                   jax.ShapeDtypeStruct((2, S), jnp.float32)],
        compiler_params=pltpu.CompilerParams(
            dimension_semantics=("arbitrary",)),
        interpret=_INTERPRET,
    )(x_A, segA, x_B, segB, G)

    H_A, H_B = pl.pallas_call(
        _final_body,
        out_shape=[jax.ShapeDtypeStruct((S, D), jnp.float32),
                   jax.ShapeDtypeStruct((S, D), jnp.float32)],
        interpret=_INTERPRET,
    )(U, d, W_gnn)
    return (H_A, H_B)
